# baseline (speedup 1.0000x reference)
import functools

import jax
import jax.numpy as jnp
from jax import lax
from jax.experimental import pallas as pl
from jax.experimental.pallas import tpu as pltpu
from jax.experimental.pallas import tpu_sc as plsc

VOCAB = 1000000
EMB = 64
BATCH = 16384
HIST = 50
TOT = BATCH * HIST
NC, NS = 2, 16
NW = NC * NS
PER_W = TOT // NW
CHUNK = 128
NCHUNK = PER_W // CHUNK
K = 4
GROUP = K * CHUNK
NGROUP = NCHUNK // K
NBT = BATCH // CHUNK
CPAD = 128

_mesh = plsc.VectorSubcoreMesh(core_axis_name="c", subcore_axis_name="s")


@functools.partial(
    pl.kernel,
    mesh=_mesh,
    out_type=jax.ShapeDtypeStruct((HIST, 8, NBT, 8, CHUNK), jnp.float32),
    compiler_params=pltpu.CompilerParams(
        use_tc_tiling_on_sc=False, needs_layout_passes=False),
    scratch_types=[
        pltpu.VMEM((NCHUNK, CHUNK), jnp.int32),
        pltpu.VMEM((GROUP, EMB), jnp.float32),
        pltpu.VMEM((GROUP, EMB), jnp.float32),
        pltpu.VMEM((8, 8, CPAD), jnp.float32),
        pltpu.VMEM((8, 8, CPAD), jnp.float32),
        pltpu.SemaphoreType.DMA,
        pltpu.SemaphoreType.DMA,
        pltpu.SemaphoreType.DMA,
        pltpu.SemaphoreType.DMA,
    ],
)
def _gather_kernel(idx_hbm, table_hbm, out_hbm, idx_v, rows0, rows1,
                   t0, t1, gsem0, gsem1, osem0, osem1):
    wid = lax.axis_index("s") * NC + lax.axis_index("c")
    base_blk = wid * NCHUNK
    pltpu.sync_copy(idx_hbm.at[pl.ds(base_blk, NCHUNK)], idx_v)

    lane = lax.iota(jnp.int32, 16)
    dts = [((d0 + lane) >> 3).astype(jnp.int32) for d0 in range(0, EMB, 16)]
    rs = [((d0 + lane) & 7).astype(jnp.int32) for d0 in range(0, EMB, 16)]
    zeros16 = jnp.zeros((16,), jnp.int32)

    def gather_descs(g, rows, gsem):
        return [
            pltpu.make_async_copy(
                table_hbm.at[idx_v.at[g * K + k]],
                rows.at[pl.ds(k * CHUNK, CHUNK)], gsem)
            for k in range(K)
        ]

    def issue(g, rows, gsem):
        for d in gather_descs(g, rows, gsem):
            d.start()

    def out_desc(blk, t, osem):
        h = blk // NBT
        bt = blk - h * NBT
        return pltpu.make_async_copy(
            t.at[:, :, pl.ds(0, CHUNK)], out_hbm.at[h, :, bt], osem)

    def transpose_block(rows, k, t):
        @plsc.parallel_loop(0, CHUNK, 1, unroll=4)
        def body(c):
            cv = zeros16 + c
            for j in range(EMB // 16):
                v = rows[k * CHUNK + c, pl.ds(j * 16, 16)]
                plsc.store_scatter(t, [dts[j], rs[j], cv], v)

    def finish(g, rows, gsem, osem, first):
        for d in gather_descs(g, rows, gsem):
            d.wait()
        for k in range(K):
            t = (t0, t1)[k % 2]
            tsem = (osem0, osem1)[k % 2]
            @pl.when(jnp.logical_not(first) | (k >= 2))
            def _():
                out_desc(0, t, tsem).wait()
            transpose_block(rows, k, t)
            out_desc(base_blk + g * K + k, t, tsem).start()

    NG2 = NGROUP // 2
    issue(0, rows0, gsem0)

    def body(s, _):
        issue(2 * s + 1, rows1, gsem1)
        finish(2 * s, rows0, gsem0, osem0, s == 0)

        @pl.when(s < NG2 - 1)
        def _():
            issue(2 * s + 2, rows0, gsem0)
        finish(2 * s + 1, rows1, gsem1, osem1, jnp.bool_(False))
        return ()

    lax.fori_loop(0, NG2, body, ())
    out_desc(0, t0, osem0).wait()
    out_desc(0, t1, osem1).wait()


def kernel(input, table):
    idx = input.T.reshape(NW * NCHUNK, CHUNK).astype(jnp.int32)
    out5 = _gather_kernel(idx, table)
    return out5.transpose(2, 4, 0, 1, 3).reshape(BATCH, HIST, EMB)


# speedup vs baseline: 1.2746x; 1.2746x over previous
import functools

import jax
import jax.numpy as jnp
from jax import lax
from jax.experimental import pallas as pl
from jax.experimental.pallas import tpu as pltpu
from jax.experimental.pallas import tpu_sc as plsc

VOCAB = 1000000
EMB = 64
BATCH = 16384
HIST = 50
TOT = BATCH * HIST
NC, NS = 2, 16
NW = NC * NS
PER_W = TOT // NW
CHUNK = 128
NCHUNK = PER_W // CHUNK
K = 4
GROUP = K * CHUNK
NGROUP = NCHUNK // K
NBT = BATCH // CHUNK
CPAD = 133

_mesh = plsc.VectorSubcoreMesh(core_axis_name="c", subcore_axis_name="s")

NBLK = (VOCAB + 127) // 128
NBLK_FULL = VOCAB // 128
BLK_W = 245
TPADW = 133


@functools.partial(
    pl.kernel,
    mesh=_mesh,
    out_type=jax.ShapeDtypeStruct((VOCAB // 2, 128), jnp.float32),
    compiler_params=pltpu.CompilerParams(needs_layout_passes=False),
    scratch_types=[
        pltpu.VMEM((EMB, 128), jnp.float32),
        pltpu.VMEM((EMB, 128), jnp.float32),
        pltpu.VMEM((EMB, TPADW), jnp.float32),
        pltpu.VMEM((EMB, TPADW), jnp.float32),
        pltpu.SemaphoreType.DMA,
        pltpu.SemaphoreType.DMA,
        pltpu.SemaphoreType.DMA,
        pltpu.SemaphoreType.DMA,
    ],
)
def _transpose_kernel(tt_hbm, tail_hbm, out_hbm, in0, in1, t0, t1,
                      isem0, isem1, osem0, osem1):
    wid = lax.axis_index("s") * NC + lax.axis_index("c")
    b_lo = wid * BLK_W

    lane = lax.iota(jnp.int32, 16)
    qvs = [((j * 16 + lane) >> 1).astype(jnp.int32) for j in range(8)]
    pvs = [(((j * 16 + lane) & 1) * EMB).astype(jnp.int32) for j in range(8)]

    def in_desc(b, buf, sem):
        return pltpu.make_async_copy(
            tt_hbm.at[:, pl.ds(b * 128, 128)], buf, sem)

    def out_desc(b, t, sem):
        return pltpu.make_async_copy(
            t.at[:, pl.ds(0, 128)], out_hbm.at[pl.ds(b * EMB, EMB)], sem)

    def transpose_full(buf, t):
        @plsc.parallel_loop(0, EMB, 1, unroll=4)
        def _(d):
            for j in range(8):
                v = buf[d, pl.ds(j * 16, 16)]
                plsc.store_scatter(t, [qvs[j], pvs[j] + d], v)

    nfull_here = jnp.minimum(jnp.maximum(NBLK_FULL - b_lo, 0), BLK_W)
    limit = b_lo + nfull_here

    def do_block(b, buf, t, isem, osem, first):
        in_desc(0, buf, isem).wait()
        @pl.when(b + 2 < limit)
        def _():
            in_desc(b + 2, buf, isem).start()
        @pl.when(jnp.logical_not(first))
        def _():
            out_desc(0, t, osem).wait()
        transpose_full(buf, t)
        out_desc(b, t, osem).start()

    @pl.when(nfull_here > 0)
    def _():
        in_desc(b_lo, in0, isem0).start()

        @pl.when(nfull_here > 1)
        def _():
            in_desc(b_lo + 1, in1, isem1).start()

        def body(s, _):
            b = b_lo + 2 * s

            @pl.when(b < limit)
            def _():
                do_block(b, in0, t0, isem0, osem0, s == 0)

            @pl.when(b + 1 < limit)
            def _():
                do_block(b + 1, in1, t1, isem1, osem1, s == 0)
            return ()

        lax.fori_loop(0, (BLK_W + 1) // 2, body, ())
        out_desc(0, t0, osem0).wait()

        @pl.when(nfull_here > 1)
        def _():
            out_desc(0, t1, osem1).wait()

    @pl.when((b_lo <= NBLK_FULL) & (NBLK_FULL < b_lo + BLK_W))
    def _():
        pltpu.sync_copy(tail_hbm, in0.at[pl.ds(0, 32)])
        pltpu.sync_copy(in0.at[pl.ds(0, 32)],
                        out_hbm.at[pl.ds(NBLK_FULL * EMB, 32)])


@functools.partial(
    pl.kernel,
    mesh=_mesh,
    out_type=jax.ShapeDtypeStruct((HIST, 8, NBT, 8, CHUNK), jnp.float32),
    compiler_params=pltpu.CompilerParams(
        use_tc_tiling_on_sc=False, needs_layout_passes=False),
    scratch_types=[
        pltpu.VMEM((NCHUNK, CHUNK), jnp.int32),
        pltpu.VMEM((GROUP, EMB), jnp.float32),
        pltpu.VMEM((GROUP, EMB), jnp.float32),
        pltpu.VMEM((8, 8, CPAD), jnp.float32),
        pltpu.VMEM((8, 8, CPAD), jnp.float32),
        pltpu.SemaphoreType.DMA,
        pltpu.SemaphoreType.DMA,
        pltpu.SemaphoreType.DMA,
        pltpu.SemaphoreType.DMA,
    ],
)
def _gather_kernel(idx_hbm, table_hbm, out_hbm, idx_v, rows0, rows1,
                   t0, t1, gsem0, gsem1, osem0, osem1):
    wid = lax.axis_index("s") * NC + lax.axis_index("c")
    base_blk = wid * NCHUNK
    pltpu.sync_copy(idx_hbm.at[pl.ds(base_blk, NCHUNK)], idx_v)

    lane = lax.iota(jnp.int32, 16)
    dts = [((d0 + lane) >> 3).astype(jnp.int32) for d0 in range(0, EMB, 16)]
    rs = [((d0 + lane) & 7).astype(jnp.int32) for d0 in range(0, EMB, 16)]
    zeros16 = jnp.zeros((16,), jnp.int32)

    def gather_descs(g, rows, gsem):
        return [
            pltpu.make_async_copy(
                table_hbm.at[idx_v.at[g * K + k]],
                rows.at[pl.ds(k * CHUNK, CHUNK)], gsem)
            for k in range(K)
        ]

    def issue(g, rows, gsem):
        for d in gather_descs(g, rows, gsem):
            d.start()

    def out_desc(blk, t, osem):
        h = blk // NBT
        bt = blk - h * NBT
        return pltpu.make_async_copy(
            t.at[:, :, pl.ds(0, CHUNK)], out_hbm.at[h, :, bt], osem)

    def transpose_block(rows, k, t):
        @plsc.parallel_loop(0, CHUNK, 1, unroll=4)
        def body(c):
            cv = zeros16 + c
            for j in range(EMB // 16):
                v = rows[k * CHUNK + c, pl.ds(j * 16, 16)]
                plsc.store_scatter(t, [dts[j], rs[j], cv], v)

    def finish(g, rows, gsem, osem, first):
        for d in gather_descs(g, rows, gsem):
            d.wait()
        for k in range(K):
            t = (t0, t1)[k % 2]
            tsem = (osem0, osem1)[k % 2]
            @pl.when(jnp.logical_not(first) | (k >= 2))
            def _():
                out_desc(0, t, tsem).wait()
            transpose_block(rows, k, t)
            out_desc(base_blk + g * K + k, t, tsem).start()

    NG2 = NGROUP // 2
    issue(0, rows0, gsem0)

    def body(s, _):
        issue(2 * s + 1, rows1, gsem1)
        finish(2 * s, rows0, gsem0, osem0, s == 0)

        @pl.when(s < NG2 - 1)
        def _():
            issue(2 * s + 2, rows0, gsem0)
        finish(2 * s + 1, rows1, gsem1, osem1, jnp.bool_(False))
        return ()

    lax.fori_loop(0, NG2, body, ())
    out_desc(0, t0, osem0).wait()
    out_desc(0, t1, osem1).wait()


def kernel(input, table):
    idx = input.T.reshape(NW * NCHUNK, CHUNK).astype(jnp.int32)
    tail = table[VOCAB - 64:].reshape(32, 128)
    table_rm = _transpose_kernel(table.T, tail).reshape(VOCAB, EMB)
    out5 = _gather_kernel(idx, table_rm)
    return out5.transpose(2, 4, 0, 1, 3).reshape(BATCH, HIST, EMB)
